# Initial kernel scaffold; baseline (speedup 1.0000x reference)
import jax
import jax.numpy as jnp
from jax.experimental import pallas as pl


def kernel(x):
    raise NotImplementedError("write your pallas kernel here")


# speedup vs baseline: 3.4184x; 3.4184x over previous
import functools
import math

import jax
import jax.numpy as jnp
from jax import lax
from jax.experimental import pallas as pl
from jax.experimental.pallas import tpu as pltpu
from jax.experimental.pallas import tpu_sc as plsc

D_MODEL = 1024
LENGTH = 8192


_ROWS_PER_BLOCK = 512


def _pe_body(out_ref):
    i = pl.program_id(0)
    shape = (_ROWS_PER_BLOCK, D_MODEL)
    pos = (lax.broadcasted_iota(jnp.int32, shape, 0)
           + (i * _ROWS_PER_BLOCK)).astype(jnp.float32)
    d = lax.broadcasted_iota(jnp.int32, shape, 1)
    dpair = d - (d % 2)
    freq = jnp.exp(dpair.astype(jnp.float32) * (-math.log(10000.0) / D_MODEL))
    arg = pos * freq
    out_ref[...] = jnp.where((d % 2) == 0, jnp.sin(arg), jnp.cos(arg))


def _build_pe():
    return pl.pallas_call(
        _pe_body,
        out_shape=jax.ShapeDtypeStruct((LENGTH, D_MODEL), jnp.float32),
        grid=(LENGTH // _ROWS_PER_BLOCK,),
        out_specs=pl.BlockSpec((_ROWS_PER_BLOCK, D_MODEL), lambda i: (i, 0)),
    )()


_NC = 2
_NS = 16
_NW = _NC * _NS
_B = 32768
_BPW = _B // _NW
_CHUNK = 64
_NCHUNK = _BPW // _CHUNK


@functools.cache
def _gather_kernel():
    mesh = plsc.VectorSubcoreMesh(core_axis_name="c", subcore_axis_name="s")

    @functools.partial(
        pl.kernel,
        mesh=mesh,
        out_type=jax.ShapeDtypeStruct((_B, D_MODEL), jnp.float32),
        scratch_types=[
            pltpu.VMEM((_CHUNK,), jnp.int32),
            pltpu.VMEM((_CHUNK, D_MODEL), jnp.float32),
            pltpu.SemaphoreType.DMA,
        ],
    )
    def gather(table_hbm, idx_hbm, out_hbm, idx_v, rows_v, sem):
        wid = lax.axis_index("s") * _NC + lax.axis_index("c")
        base = wid * _BPW

        def body(c, carry):
            off = base + c * _CHUNK
            pltpu.sync_copy(idx_hbm.at[pl.ds(off, _CHUNK)], idx_v)
            pltpu.async_copy(table_hbm.at[idx_v], rows_v, sem).wait()
            pltpu.sync_copy(rows_v, out_hbm.at[pl.ds(off, _CHUNK)])
            return carry

        lax.fori_loop(0, _NCHUNK, body, 0)

    return gather


def kernel(x):
    pe = _build_pe()
    flat = x.reshape(-1)
    out = _gather_kernel()(pe, flat)
    return out.reshape(x.shape + (D_MODEL,))


# speedup vs baseline: 3.7310x; 1.0914x over previous
import functools
import math

import jax
import jax.numpy as jnp
from jax import lax
from jax.experimental import pallas as pl
from jax.experimental.pallas import tpu as pltpu
from jax.experimental.pallas import tpu_sc as plsc

D_MODEL = 1024
LENGTH = 8192


_ROWS_PER_BLOCK = 512


def _pe_body(out_ref):
    i = pl.program_id(0)
    shape = (_ROWS_PER_BLOCK, D_MODEL)
    pos = (lax.broadcasted_iota(jnp.int32, shape, 0)
           + (i * _ROWS_PER_BLOCK)).astype(jnp.float32)
    d = lax.broadcasted_iota(jnp.int32, shape, 1)
    odd = (d % 2).astype(jnp.float32)
    dpair = d - (d % 2)
    freq = jnp.exp(dpair.astype(jnp.float32) * (-math.log(10000.0) / D_MODEL))
    out_ref[...] = jnp.sin(pos * freq + odd * (math.pi / 2))


def _build_pe():
    return pl.pallas_call(
        _pe_body,
        out_shape=jax.ShapeDtypeStruct((LENGTH, D_MODEL), jnp.float32),
        grid=(LENGTH // _ROWS_PER_BLOCK,),
        out_specs=pl.BlockSpec((_ROWS_PER_BLOCK, D_MODEL), lambda i: (i, 0)),
    )()


_NC = 2
_NS = 16
_NW = _NC * _NS
_B = 32768
_BPW = _B // _NW
_CHUNK = 32
_NCHUNK = _BPW // _CHUNK


@functools.cache
def _gather_kernel():
    mesh = plsc.VectorSubcoreMesh(core_axis_name="c", subcore_axis_name="s")

    @functools.partial(
        pl.kernel,
        mesh=mesh,
        out_type=jax.ShapeDtypeStruct((_B, D_MODEL), jnp.float32),
        scratch_types=[
            pltpu.VMEM((_BPW,), jnp.int32),
            pltpu.VMEM((_CHUNK, D_MODEL), jnp.float32),
            pltpu.VMEM((_CHUNK, D_MODEL), jnp.float32),
            pltpu.SemaphoreType.DMA,
            pltpu.SemaphoreType.DMA,
        ],
    )
    def gather(table_hbm, idx_hbm, out_hbm, idx_all, buf0, buf1, sem0, sem1):
        wid = lax.axis_index("s") * _NC + lax.axis_index("c")
        base = wid * _BPW
        pltpu.sync_copy(idx_hbm.at[pl.ds(base, _BPW)], idx_all)

        bufs = (buf0, buf1)
        sems = (sem0, sem1)

        def copy(c, b):
            return pltpu.make_async_copy(
                table_hbm.at[idx_all.at[pl.ds(c * _CHUNK, _CHUNK)]],
                bufs[b], sems[b])

        copy(0, 0).start()

        def body(c0, carry):
            for b in range(2):
                c = c0 * 2 + b
                nxt = jnp.minimum(c + 1, _NCHUNK - 1)
                copy(nxt, (b + 1) % 2).start()
                copy(c, b).wait()
                pltpu.sync_copy(bufs[b], out_hbm.at[pl.ds(base + c * _CHUNK, _CHUNK)])
            return carry

        lax.fori_loop(0, _NCHUNK // 2, body, 0)
        copy(_NCHUNK - 1, 0).wait()

    return gather


def kernel(x):
    pe = _build_pe()
    flat = x.reshape(-1)
    out = _gather_kernel()(pe, flat)
    return out.reshape(x.shape + (D_MODEL,))


# speedup vs baseline: 5.7505x; 1.5413x over previous
import functools
import math

import jax
import jax.numpy as jnp
from jax import lax
from jax.experimental import pallas as pl
from jax.experimental.pallas import tpu as pltpu
from jax.experimental.pallas import tpu_sc as plsc

D_MODEL = 1024
LENGTH = 8192


_ROWS_PER_BLOCK = 512


def _freq(d):
    dpair = d - (d % 2)
    return jnp.exp(dpair.astype(jnp.float32) * (-math.log(10000.0) / D_MODEL))


def _lo_body(cl_ref, sl_ref):
    shape = (_ROWS_PER_BLOCK, D_MODEL)
    r = lax.broadcasted_iota(jnp.int32, shape, 0).astype(jnp.float32)
    d = lax.broadcasted_iota(jnp.int32, shape, 1)
    arg = r * _freq(d)
    sl_ref[...] = jnp.sin(arg)
    cl_ref[...] = jnp.sin(arg + math.pi / 2)


def _pe_body(cl_ref, sl_ref, out_ref):
    i = pl.program_id(0)
    shape1 = (1, D_MODEL)
    d1 = lax.broadcasted_iota(jnp.int32, shape1, 1)
    bf = (i * _ROWS_PER_BLOCK).astype(jnp.float32) * _freq(d1)
    sbf = jnp.sin(bf)
    cbf = jnp.sin(bf + math.pi / 2)
    even1 = (d1 % 2) == 0
    u = jnp.where(even1, sbf, cbf)
    v = jnp.where(even1, cbf, -sbf)
    out_ref[...] = u * cl_ref[...] + v * sl_ref[...]


def _build_pe():
    cl, sl = pl.pallas_call(
        _lo_body,
        out_shape=[
            jax.ShapeDtypeStruct((_ROWS_PER_BLOCK, D_MODEL), jnp.float32),
            jax.ShapeDtypeStruct((_ROWS_PER_BLOCK, D_MODEL), jnp.float32),
        ],
    )()
    return pl.pallas_call(
        _pe_body,
        out_shape=jax.ShapeDtypeStruct((LENGTH, D_MODEL), jnp.float32),
        grid=(LENGTH // _ROWS_PER_BLOCK,),
        in_specs=[
            pl.BlockSpec((_ROWS_PER_BLOCK, D_MODEL), lambda i: (0, 0)),
            pl.BlockSpec((_ROWS_PER_BLOCK, D_MODEL), lambda i: (0, 0)),
        ],
        out_specs=pl.BlockSpec((_ROWS_PER_BLOCK, D_MODEL), lambda i: (i, 0)),
    )(cl, sl)


_NC = 2
_NS = 16
_NW = _NC * _NS
_B = 32768
_BPW = _B // _NW
_CHUNK = 32
_NCHUNK = _BPW // _CHUNK


@functools.cache
def _gather_kernel():
    mesh = plsc.VectorSubcoreMesh(core_axis_name="c", subcore_axis_name="s")

    @functools.partial(
        pl.kernel,
        mesh=mesh,
        out_type=jax.ShapeDtypeStruct((_B, D_MODEL), jnp.float32),
        scratch_types=[
            pltpu.VMEM((_BPW,), jnp.int32),
            pltpu.VMEM((_CHUNK, D_MODEL), jnp.float32),
            pltpu.VMEM((_CHUNK, D_MODEL), jnp.float32),
            pltpu.SemaphoreType.DMA,
            pltpu.SemaphoreType.DMA,
        ],
    )
    def gather(table_hbm, idx_hbm, out_hbm, idx_all, buf0, buf1, sem0, sem1):
        wid = lax.axis_index("s") * _NC + lax.axis_index("c")
        base = wid * _BPW
        pltpu.sync_copy(idx_hbm.at[pl.ds(base, _BPW)], idx_all)

        bufs = (buf0, buf1)
        sems = (sem0, sem1)

        def copy(c, b):
            return pltpu.make_async_copy(
                table_hbm.at[idx_all.at[pl.ds(c * _CHUNK, _CHUNK)]],
                bufs[b], sems[b])

        copy(0, 0).start()

        def body(c0, carry):
            for b in range(2):
                c = c0 * 2 + b
                nxt = jnp.minimum(c + 1, _NCHUNK - 1)
                copy(nxt, (b + 1) % 2).start()
                copy(c, b).wait()
                pltpu.sync_copy(bufs[b], out_hbm.at[pl.ds(base + c * _CHUNK, _CHUNK)])
            return carry

        lax.fori_loop(0, _NCHUNK // 2, body, 0)
        copy(_NCHUNK - 1, 0).wait()

    return gather


def kernel(x):
    pe = _build_pe()
    flat = x.reshape(-1)
    out = _gather_kernel()(pe, flat)
    return out.reshape(x.shape + (D_MODEL,))


# speedup vs baseline: 6.4311x; 1.1184x over previous
import functools
import math

import jax
import jax.numpy as jnp
from jax import lax
from jax.experimental import pallas as pl
from jax.experimental.pallas import tpu as pltpu
from jax.experimental.pallas import tpu_sc as plsc

D_MODEL = 1024
LENGTH = 8192


_ROWS_PER_BLOCK = 512


def _freq(d):
    dpair = d - (d % 2)
    return jnp.exp(dpair.astype(jnp.float32) * (-math.log(10000.0) / D_MODEL))


def _pe_body(out_ref, cl_ref, sl_ref):
    i = pl.program_id(0)

    @pl.when(i == 0)
    def _init():
        shape_a = (16, D_MODEL)
        shape_b = (32, D_MODEL)
        arga = (lax.broadcasted_iota(jnp.int32, shape_a, 0).astype(jnp.float32)
                * 32.0) * _freq(lax.broadcasted_iota(jnp.int32, shape_a, 1))
        argb = (lax.broadcasted_iota(jnp.int32, shape_b, 0).astype(jnp.float32)
                * _freq(lax.broadcasted_iota(jnp.int32, shape_b, 1)))
        sa = jnp.sin(arga)
        ca = jnp.sin(arga + math.pi / 2)
        sb = jnp.sin(argb)
        cb = jnp.sin(argb + math.pi / 2)
        for a in range(16):
            caa = ca[a:a + 1, :]
            saa = sa[a:a + 1, :]
            cl_ref[pl.ds(32 * a, 32), :] = caa * cb - saa * sb
            sl_ref[pl.ds(32 * a, 32), :] = saa * cb + caa * sb

    shape1 = (1, D_MODEL)
    d1 = lax.broadcasted_iota(jnp.int32, shape1, 1)
    bf = (i * _ROWS_PER_BLOCK).astype(jnp.float32) * _freq(d1)
    sbf = jnp.sin(bf)
    cbf = jnp.sin(bf + math.pi / 2)
    even1 = (d1 % 2) == 0
    u = jnp.where(even1, sbf, cbf)
    v = jnp.where(even1, cbf, -sbf)
    out_ref[...] = u * cl_ref[...] + v * sl_ref[...]


def _build_pe():
    return pl.pallas_call(
        _pe_body,
        out_shape=jax.ShapeDtypeStruct((LENGTH, D_MODEL), jnp.float32),
        grid=(LENGTH // _ROWS_PER_BLOCK,),
        out_specs=pl.BlockSpec((_ROWS_PER_BLOCK, D_MODEL), lambda i: (i, 0)),
        scratch_shapes=[
            pltpu.VMEM((_ROWS_PER_BLOCK, D_MODEL), jnp.float32),
            pltpu.VMEM((_ROWS_PER_BLOCK, D_MODEL), jnp.float32),
        ],
    )()


_NC = 2
_NS = 16
_NW = _NC * _NS
_B = 32768
_BPW = _B // _NW
_CHUNK = 32
_NCHUNK = _BPW // _CHUNK


@functools.cache
def _gather_kernel():
    mesh = plsc.VectorSubcoreMesh(core_axis_name="c", subcore_axis_name="s")

    @functools.partial(
        pl.kernel,
        mesh=mesh,
        out_type=jax.ShapeDtypeStruct((_B, D_MODEL), jnp.float32),
        scratch_types=[
            pltpu.VMEM((_BPW,), jnp.int32),
            pltpu.VMEM((_CHUNK, D_MODEL), jnp.float32),
            pltpu.VMEM((_CHUNK, D_MODEL), jnp.float32),
            pltpu.SemaphoreType.DMA,
            pltpu.SemaphoreType.DMA,
        ],
    )
    def gather(table_hbm, idx_hbm, out_hbm, idx_all, buf0, buf1, sem0, sem1):
        wid = lax.axis_index("s") * _NC + lax.axis_index("c")
        base = wid * _BPW
        pltpu.sync_copy(idx_hbm.at[pl.ds(base, _BPW)], idx_all)

        bufs = (buf0, buf1)
        sems = (sem0, sem1)

        def copy(c, b):
            return pltpu.make_async_copy(
                table_hbm.at[idx_all.at[pl.ds(c * _CHUNK, _CHUNK)]],
                bufs[b], sems[b])

        copy(0, 0).start()

        def body(c0, carry):
            for b in range(2):
                c = c0 * 2 + b
                nxt = jnp.minimum(c + 1, _NCHUNK - 1)
                copy(nxt, (b + 1) % 2).start()
                copy(c, b).wait()
                pltpu.sync_copy(bufs[b], out_hbm.at[pl.ds(base + c * _CHUNK, _CHUNK)])
            return carry

        lax.fori_loop(0, _NCHUNK // 2, body, 0)
        copy(_NCHUNK - 1, 0).wait()

    return gather


def kernel(x):
    pe = _build_pe()
    flat = x.reshape(-1)
    out = _gather_kernel()(pe, flat)
    return out.reshape(x.shape + (D_MODEL,))


# speedup vs baseline: 6.4802x; 1.0076x over previous
import functools
import math

import jax
import jax.numpy as jnp
from jax import lax
from jax.experimental import pallas as pl
from jax.experimental.pallas import tpu as pltpu
from jax.experimental.pallas import tpu_sc as plsc

D_MODEL = 1024
LENGTH = 8192


_ROWS_PER_BLOCK = 512


def _freq(d):
    dpair = d - (d % 2)
    return jnp.exp(dpair.astype(jnp.float32) * (-math.log(10000.0) / D_MODEL))


def _pe_body(out_ref, cl_ref, sl_ref):
    i = pl.program_id(0)

    @pl.when(i == 0)
    def _init():
        shape_a = (16, D_MODEL)
        shape_b = (32, D_MODEL)
        arga = (lax.broadcasted_iota(jnp.int32, shape_a, 0).astype(jnp.float32)
                * 32.0) * _freq(lax.broadcasted_iota(jnp.int32, shape_a, 1))
        argb = (lax.broadcasted_iota(jnp.int32, shape_b, 0).astype(jnp.float32)
                * _freq(lax.broadcasted_iota(jnp.int32, shape_b, 1)))
        sa = jnp.sin(arga)
        ca = jnp.sin(arga + math.pi / 2)
        sb = jnp.sin(argb)
        cb = jnp.sin(argb + math.pi / 2)
        for a in range(16):
            caa = ca[a:a + 1, :]
            saa = sa[a:a + 1, :]
            cl_ref[pl.ds(32 * a, 32), :] = caa * cb - saa * sb
            sl_ref[pl.ds(32 * a, 32), :] = saa * cb + caa * sb

    shape1 = (1, D_MODEL)
    d1 = lax.broadcasted_iota(jnp.int32, shape1, 1)
    bf = (i * _ROWS_PER_BLOCK).astype(jnp.float32) * _freq(d1)
    sbf = jnp.sin(bf)
    cbf = jnp.sin(bf + math.pi / 2)
    even1 = (d1 % 2) == 0
    u = jnp.where(even1, sbf, cbf)
    v = jnp.where(even1, cbf, -sbf)
    out_ref[...] = u * cl_ref[...] + v * sl_ref[...]


def _build_pe():
    return pl.pallas_call(
        _pe_body,
        out_shape=jax.ShapeDtypeStruct((LENGTH, D_MODEL), jnp.float32),
        grid=(LENGTH // _ROWS_PER_BLOCK,),
        out_specs=pl.BlockSpec((_ROWS_PER_BLOCK, D_MODEL), lambda i: (i, 0)),
        scratch_shapes=[
            pltpu.VMEM((_ROWS_PER_BLOCK, D_MODEL), jnp.float32),
            pltpu.VMEM((_ROWS_PER_BLOCK, D_MODEL), jnp.float32),
        ],
    )()


_NC = 2
_NS = 16
_NW = _NC * _NS
_B = 32768
_BPW = _B // _NW
_CHUNK = 32
_NCHUNK = _BPW // _CHUNK


@functools.cache
def _gather_kernel():
    mesh = plsc.VectorSubcoreMesh(core_axis_name="c", subcore_axis_name="s")

    @functools.partial(
        pl.kernel,
        mesh=mesh,
        out_type=jax.ShapeDtypeStruct((_B, D_MODEL), jnp.float32),
        scratch_types=[
            pltpu.VMEM((_BPW,), jnp.int32),
            pltpu.VMEM((_CHUNK, D_MODEL), jnp.float32),
            pltpu.VMEM((_CHUNK, D_MODEL), jnp.float32),
            pltpu.SemaphoreType.DMA,
            pltpu.SemaphoreType.DMA,
            pltpu.SemaphoreType.DMA,
            pltpu.SemaphoreType.DMA,
        ],
    )
    def gather(table_hbm, idx_hbm, out_hbm, idx_all, buf0, buf1,
               gsem0, gsem1, wsem0, wsem1):
        wid = lax.axis_index("s") * _NC + lax.axis_index("c")
        base = wid * _BPW
        pltpu.sync_copy(idx_hbm.at[pl.ds(base, _BPW)], idx_all)

        bufs = (buf0, buf1)
        gsems = (gsem0, gsem1)
        wsems = (wsem0, wsem1)

        def gcopy(c, b):
            return pltpu.make_async_copy(
                table_hbm.at[idx_all.at[pl.ds(c * _CHUNK, _CHUNK)]],
                bufs[b], gsems[b])

        def wcopy(c, b):
            return pltpu.make_async_copy(
                bufs[b], out_hbm.at[pl.ds(base + c * _CHUNK, _CHUNK)],
                wsems[b])

        gcopy(0, 0).start()
        gcopy(1, 1).start()
        gcopy(0, 0).wait()
        wcopy(0, 0).start()

        def body(c0, carry):
            for j in range(2):
                c = c0 * 2 + 1 + j
                b = (1 + j) % 2
                wcopy(c - 1, (b + 1) % 2).wait()
                gcopy(c + 1, (b + 1) % 2).start()
                gcopy(c, b).wait()
                wcopy(c, b).start()
            return carry

        lax.fori_loop(0, (_NCHUNK - 2) // 2, body, 0)

        c_last = _NCHUNK - 1
        b_last = c_last % 2
        wcopy(c_last - 1, (b_last + 1) % 2).wait()
        gcopy(c_last, b_last).wait()
        wcopy(c_last, b_last).start()
        wcopy(c_last, b_last).wait()

    return gather


def kernel(x):
    pe = _build_pe()
    flat = x.reshape(-1)
    out = _gather_kernel()(pe, flat)
    return out.reshape(x.shape + (D_MODEL,))
